# baseline (speedup 1.0000x reference)
import functools

import jax
import jax.numpy as jnp
from jax import lax
from jax.experimental import pallas as pl
from jax.experimental.pallas import tpu as pltpu
from jax.experimental.pallas import tpu_sc as plsc

B = 4096
D = 64
NROWS = 1000000
L = 16
NC = 2
NS = 16
NW = NC * NS
BPW = B // NW
NG = BPW // L
W = 128
WMAX = NROWS - W

RB = 512


def _sc_scores(user, pos_item, neg_item, user_table_t, item_table_t):
    BS = user.shape[0]
    BPW = BS // NW
    NG = BPW // L
    mesh = plsc.VectorSubcoreMesh(core_axis_name="c", subcore_axis_name="s")

    @functools.partial(
        pl.kernel,
        mesh=mesh,
        compiler_params=pltpu.CompilerParams(needs_layout_passes=False),
        out_type=[
            jax.ShapeDtypeStruct((BS,), jnp.float32),
            jax.ShapeDtypeStruct((BS,), jnp.float32),
        ],
        scratch_types=[
            pltpu.VMEM((BPW,), jnp.int32),
            pltpu.VMEM((BPW,), jnp.int32),
            pltpu.VMEM((BPW,), jnp.int32),
            pltpu.VMEM((D, W), jnp.float32),
            pltpu.VMEM((D, W), jnp.float32),
            pltpu.VMEM((D, W), jnp.float32),
            pltpu.VMEM((D, W), jnp.float32),
            pltpu.VMEM((D, W), jnp.float32),
            pltpu.VMEM((D, W), jnp.float32),
            pltpu.VMEM((D, W), jnp.float32),
            pltpu.VMEM((D, W), jnp.float32),
            pltpu.VMEM((D, W), jnp.float32),
            pltpu.VMEM((BPW,), jnp.float32),
            pltpu.VMEM((BPW,), jnp.float32),
            pltpu.SemaphoreType.DMA,
            pltpu.SemaphoreType.DMA,
            pltpu.SemaphoreType.DMA,
        ],
    )
    def k(u_hbm, p_hbm, n_hbm, ut_hbm, it_hbm, ps_out, ns_out,
          uidx, pidx, nidx, wu0, wu1, wu2, wp0, wp1, wp2, wn0, wn1, wn2,
          ps_v, ns_v, sem0, sem1, sem2):
        wid = lax.axis_index("s") * NC + lax.axis_index("c")
        base = wid * BPW
        pltpu.sync_copy(u_hbm.at[pl.ds(base, BPW)], uidx)
        pltpu.sync_copy(p_hbm.at[pl.ds(base, BPW)], pidx)
        pltpu.sync_copy(n_hbm.at[pl.ds(base, BPW)], nidx)

        lane = lax.iota(jnp.int32, L)
        rows4 = [lax.iota(jnp.int32, L) + c * L for c in range(D // L)]

        def win_off(i):
            return i & jnp.int32(-W)

        bufs = ((wu0, wp0, wn0, sem0), (wu1, wp1, wn1, sem1),
                (wu2, wp2, wn2, sem2))
        NBUF = len(bufs)

        def group(g, carry):
            gbase = g * L
            uvec = uidx[pl.ds(gbase, L)]
            pvec = pidx[pl.ds(gbase, L)]
            nvec = nidx[pl.ds(gbase, L)]

            def issue(jj):
                bu, bp, bn, sem = bufs[jj % NBUF]
                iu, ip, iN = uvec[jj], pvec[jj], nvec[jj]
                ou, op, oN = win_off(iu), win_off(ip), win_off(iN)
                cs = (
                    pltpu.async_copy(
                        ut_hbm.at[:, pl.ds(pl.multiple_of(ou, W), W)], bu, sem),
                    pltpu.async_copy(
                        it_hbm.at[:, pl.ds(pl.multiple_of(op, W), W)], bp, sem),
                    pltpu.async_copy(
                        it_hbm.at[:, pl.ds(pl.multiple_of(oN, W), W)], bn, sem),
                )
                return cs, (iu - ou, ip - op, iN - oN)

            psel = jnp.zeros((L,), jnp.float32)
            nsel = jnp.zeros((L,), jnp.float32)
            inflight = [issue(0), issue(1)]
            for j in range(L):
                if j + 2 < L:
                    inflight.append(issue(j + 2))
                cs, (lu_s, lp_s, ln_s) = inflight.pop(0)
                for c_ in cs:
                    c_.wait()
                bu, bp, bn, _ = bufs[j % NBUF]
                lu = jnp.full((L,), lu_s, jnp.int32)
                lp = jnp.full((L,), lp_s, jnp.int32)
                ln = jnp.full((L,), ln_s, jnp.int32)
                pacc = jnp.zeros((L,), jnp.float32)
                nacc = jnp.zeros((L,), jnp.float32)
                for c in range(D // L):
                    uv = plsc.load_gather(bu, [rows4[c], lu])
                    pv = plsc.load_gather(bp, [rows4[c], lp])
                    nv = plsc.load_gather(bn, [rows4[c], ln])
                    pacc = pacc + uv * pv
                    nacc = nacc + uv * nv
                mask = lane == j
                psel = jnp.where(mask, jnp.sum(pacc), psel)
                nsel = jnp.where(mask, jnp.sum(nacc), nsel)
            ps_v[pl.ds(gbase, L)] = psel
            ns_v[pl.ds(gbase, L)] = nsel
            return carry

        lax.fori_loop(0, NG, group, 0)
        pltpu.sync_copy(ps_v, ps_out.at[pl.ds(base, BPW)])
        pltpu.sync_copy(ns_v, ns_out.at[pl.ds(base, BPW)])

    return k(user, pos_item, neg_item, user_table_t, item_table_t)


K = 8


def _tc_scores(user, pos_item, neg_item, user_table_t, item_table_t):
    S = user.shape[0]

    def body(uref, pref, nref, *refs):
        ubs, pbs, nbs = refs[0:K], refs[K:2 * K], refs[2 * K:3 * K]
        po, no = refs[3 * K], refs[3 * K + 1]
        i = pl.program_id(0)
        lanes = lax.broadcasted_iota(jnp.int32, (1, W), 1)
        pouts, nouts = [], []
        for k in range(K):
            r = i * K + k
            lu, lp, ln = uref[r] & (W - 1), pref[r] & (W - 1), nref[r] & (W - 1)
            uv = jnp.sum(jnp.where(lanes == lu, ubs[k][...], 0.0),
                         axis=1, keepdims=True)
            pv = jnp.sum(jnp.where(lanes == lp, pbs[k][...], 0.0),
                         axis=1, keepdims=True)
            nv = jnp.sum(jnp.where(lanes == ln, nbs[k][...], 0.0),
                         axis=1, keepdims=True)
            pouts.append(jnp.reshape(jnp.sum(uv * pv), (1, 1)))
            nouts.append(jnp.reshape(jnp.sum(uv * nv), (1, 1)))
        po[...] = jnp.concatenate(pouts, axis=0)
        no[...] = jnp.concatenate(nouts, axis=0)

    def uspec(k):
        return pl.BlockSpec((D, W), lambda i, u, p, n, k=k: (0, u[i * K + k] // W))

    def pspec(k):
        return pl.BlockSpec((D, W), lambda i, u, p, n, k=k: (0, p[i * K + k] // W))

    def nspec(k):
        return pl.BlockSpec((D, W), lambda i, u, p, n, k=k: (0, n[i * K + k] // W))

    grid_spec = pltpu.PrefetchScalarGridSpec(
        num_scalar_prefetch=3,
        grid=(S // K,),
        in_specs=([uspec(k) for k in range(K)]
                  + [pspec(k) for k in range(K)]
                  + [nspec(k) for k in range(K)]),
        out_specs=[
            pl.BlockSpec((K, 1), lambda i, u, p, n: (i, 0)),
            pl.BlockSpec((K, 1), lambda i, u, p, n: (i, 0)),
        ],
    )
    ops = ([user_table_t] * K) + ([item_table_t] * (2 * K))
    return pl.pallas_call(
        body,
        grid_spec=grid_spec,
        out_shape=[jax.ShapeDtypeStruct((S, 1), jnp.float32)] * 2,
    )(user, pos_item, neg_item, *ops)


def _tc_loss_sum(p, n):
    p2 = p.reshape(B, 1)
    n2 = n.reshape(1, B)

    def body(p_ref, n_ref, out_ref):
        i = pl.program_id(0)
        z = n_ref[...] - p_ref[...]
        m = jnp.maximum(z, 0.0)
        t = 1.0 + jnp.exp(-jnp.abs(z))
        acc = t[:, 0:128]
        for c in range(1, B // 128):
            acc = acc * t[:, c * 128:(c + 1) * 128]
        part = jnp.sum(m) + jnp.sum(jnp.log(acc))

        @pl.when(i == 0)
        def _():
            out_ref[...] = jnp.zeros_like(out_ref)

        out_ref[...] += jnp.reshape(part, (1, 1))

    out = pl.pallas_call(
        body,
        grid=(B // RB,),
        in_specs=[
            pl.BlockSpec((RB, 1), lambda i: (i, 0)),
            pl.BlockSpec((1, B), lambda i: (0, 0)),
        ],
        out_specs=pl.BlockSpec((1, 1), lambda i: (0, 0)),
        out_shape=jax.ShapeDtypeStruct((1, 1), jnp.float32),
    )(p2, n2)
    return out[0, 0]


B_SC = 2048


def kernel(user, pos_item, neg_item, user_table, item_table):
    nflat = jnp.reshape(neg_item, (-1,))
    ttu, tti = user_table.T, item_table.T
    p_sc, n_sc = _sc_scores(user[:B_SC], pos_item[:B_SC], nflat[:B_SC],
                            ttu, tti)
    p_tc, n_tc = _tc_scores(user[B_SC:], pos_item[B_SC:], nflat[B_SC:],
                            ttu, tti)
    p = jnp.concatenate([p_sc, p_tc[:, 0]])
    nvec = jnp.concatenate([n_sc, n_tc[:, 0]])
    loss = _tc_loss_sum(p, nvec) / (B * B)
    logits = p.reshape(B, 1)
    return (logits, loss)


# speedup vs baseline: 1.7990x; 1.7990x over previous
import functools

import jax
import jax.numpy as jnp
from jax import lax
from jax.experimental import pallas as pl
from jax.experimental.pallas import tpu as pltpu
from jax.experimental.pallas import tpu_sc as plsc

B = 4096
D = 64
NROWS = 1000000
L = 16
NC = 2
NS = 16
NW = NC * NS
BPW = B // NW
NG = BPW // L
W = 128
WMAX = NROWS - W

RB = 512


def _sc_scores(user, pos_item, neg_item, user_table_t, item_table_t):
    BS = user.shape[0]
    BPW = BS // NW
    NG = BPW // L
    mesh = plsc.VectorSubcoreMesh(core_axis_name="c", subcore_axis_name="s")

    @functools.partial(
        pl.kernel,
        mesh=mesh,
        compiler_params=pltpu.CompilerParams(needs_layout_passes=False),
        out_type=[
            jax.ShapeDtypeStruct((BS,), jnp.float32),
            jax.ShapeDtypeStruct((BS,), jnp.float32),
        ],
        scratch_types=[
            pltpu.VMEM((BPW,), jnp.int32),
            pltpu.VMEM((BPW,), jnp.int32),
            pltpu.VMEM((BPW,), jnp.int32),
            pltpu.VMEM((D, W), jnp.float32),
            pltpu.VMEM((D, W), jnp.float32),
            pltpu.VMEM((D, W), jnp.float32),
            pltpu.VMEM((D, W), jnp.float32),
            pltpu.VMEM((D, W), jnp.float32),
            pltpu.VMEM((D, W), jnp.float32),
            pltpu.VMEM((D, W), jnp.float32),
            pltpu.VMEM((D, W), jnp.float32),
            pltpu.VMEM((D, W), jnp.float32),
            pltpu.VMEM((BPW,), jnp.float32),
            pltpu.VMEM((BPW,), jnp.float32),
            pltpu.SemaphoreType.DMA,
            pltpu.SemaphoreType.DMA,
            pltpu.SemaphoreType.DMA,
        ],
    )
    def k(u_hbm, p_hbm, n_hbm, ut_hbm, it_hbm, ps_out, ns_out,
          uidx, pidx, nidx, wu0, wu1, wu2, wp0, wp1, wp2, wn0, wn1, wn2,
          ps_v, ns_v, sem0, sem1, sem2):
        wid = lax.axis_index("s") * NC + lax.axis_index("c")
        base = wid * BPW
        pltpu.sync_copy(u_hbm.at[pl.ds(base, BPW)], uidx)
        pltpu.sync_copy(p_hbm.at[pl.ds(base, BPW)], pidx)
        pltpu.sync_copy(n_hbm.at[pl.ds(base, BPW)], nidx)

        lane = lax.iota(jnp.int32, L)
        rows4 = [lax.iota(jnp.int32, L) + c * L for c in range(D // L)]

        def win_off(i):
            return i & jnp.int32(-W)

        bufs = ((wu0, wp0, wn0, sem0), (wu1, wp1, wn1, sem1),
                (wu2, wp2, wn2, sem2))
        NBUF = len(bufs)

        def group(g, carry):
            gbase = g * L
            uvec = uidx[pl.ds(gbase, L)]
            pvec = pidx[pl.ds(gbase, L)]
            nvec = nidx[pl.ds(gbase, L)]

            def issue(jj):
                bu, bp, bn, sem = bufs[jj % NBUF]
                iu, ip, iN = uvec[jj], pvec[jj], nvec[jj]
                ou, op, oN = win_off(iu), win_off(ip), win_off(iN)
                cs = (
                    pltpu.async_copy(
                        ut_hbm.at[:, pl.ds(pl.multiple_of(ou, W), W)], bu, sem),
                    pltpu.async_copy(
                        it_hbm.at[:, pl.ds(pl.multiple_of(op, W), W)], bp, sem),
                    pltpu.async_copy(
                        it_hbm.at[:, pl.ds(pl.multiple_of(oN, W), W)], bn, sem),
                )
                return cs, (iu - ou, ip - op, iN - oN)

            psel = jnp.zeros((L,), jnp.float32)
            nsel = jnp.zeros((L,), jnp.float32)
            inflight = [issue(0), issue(1)]
            for j in range(L):
                if j + 2 < L:
                    inflight.append(issue(j + 2))
                cs, (lu_s, lp_s, ln_s) = inflight.pop(0)
                for c_ in cs:
                    c_.wait()
                bu, bp, bn, _ = bufs[j % NBUF]
                lu = jnp.full((L,), lu_s, jnp.int32)
                lp = jnp.full((L,), lp_s, jnp.int32)
                ln = jnp.full((L,), ln_s, jnp.int32)
                pacc = jnp.zeros((L,), jnp.float32)
                nacc = jnp.zeros((L,), jnp.float32)
                for c in range(D // L):
                    uv = plsc.load_gather(bu, [rows4[c], lu])
                    pv = plsc.load_gather(bp, [rows4[c], lp])
                    nv = plsc.load_gather(bn, [rows4[c], ln])
                    pacc = pacc + uv * pv
                    nacc = nacc + uv * nv
                mask = lane == j
                psel = jnp.where(mask, jnp.sum(pacc), psel)
                nsel = jnp.where(mask, jnp.sum(nacc), nsel)
            ps_v[pl.ds(gbase, L)] = psel
            ns_v[pl.ds(gbase, L)] = nsel
            return carry

        lax.fori_loop(0, NG, group, 0)
        pltpu.sync_copy(ps_v, ps_out.at[pl.ds(base, BPW)])
        pltpu.sync_copy(ns_v, ns_out.at[pl.ds(base, BPW)])

    return k(user, pos_item, neg_item, user_table_t, item_table_t)


K = 8


def _tc_scores(user, pos_item, neg_item, user_table_t, item_table_t):
    S = user.shape[0]

    def body(uref, pref, nref, *refs):
        ubs, pbs, nbs = refs[0:K], refs[K:2 * K], refs[2 * K:3 * K]
        po, no = refs[3 * K], refs[3 * K + 1]
        i = pl.program_id(0)
        lanes = lax.broadcasted_iota(jnp.int32, (1, W), 1)
        pouts, nouts = [], []
        for k in range(K):
            r = i * K + k
            lu, lp, ln = uref[r] & (W - 1), pref[r] & (W - 1), nref[r] & (W - 1)
            uv = jnp.sum(jnp.where(lanes == lu, ubs[k][...], 0.0),
                         axis=1, keepdims=True)
            pv = jnp.sum(jnp.where(lanes == lp, pbs[k][...], 0.0),
                         axis=1, keepdims=True)
            nv = jnp.sum(jnp.where(lanes == ln, nbs[k][...], 0.0),
                         axis=1, keepdims=True)
            pouts.append(jnp.reshape(jnp.sum(uv * pv), (1, 1)))
            nouts.append(jnp.reshape(jnp.sum(uv * nv), (1, 1)))
        po[...] = jnp.concatenate(pouts, axis=0)
        no[...] = jnp.concatenate(nouts, axis=0)

    def uspec(k):
        return pl.BlockSpec((D, W), lambda i, u, p, n, k=k: (0, u[i * K + k] // W))

    def pspec(k):
        return pl.BlockSpec((D, W), lambda i, u, p, n, k=k: (0, p[i * K + k] // W))

    def nspec(k):
        return pl.BlockSpec((D, W), lambda i, u, p, n, k=k: (0, n[i * K + k] // W))

    grid_spec = pltpu.PrefetchScalarGridSpec(
        num_scalar_prefetch=3,
        grid=(S // K,),
        in_specs=([uspec(k) for k in range(K)]
                  + [pspec(k) for k in range(K)]
                  + [nspec(k) for k in range(K)]),
        out_specs=[
            pl.BlockSpec((K, 1), lambda i, u, p, n: (i, 0)),
            pl.BlockSpec((K, 1), lambda i, u, p, n: (i, 0)),
        ],
    )
    ops = ([user_table_t] * K) + ([item_table_t] * (2 * K))
    return pl.pallas_call(
        body,
        grid_spec=grid_spec,
        out_shape=[jax.ShapeDtypeStruct((S, 1), jnp.float32)] * 2,
    )(user, pos_item, neg_item, *ops)


def _tc_loss_sum(p, n):
    p2 = p.reshape(B, 1)
    n2 = n.reshape(1, B)

    def body(p_ref, n_ref, out_ref):
        i = pl.program_id(0)
        z = n_ref[...] - p_ref[...]
        m = jnp.maximum(z, 0.0)
        t = 1.0 + jnp.exp(-jnp.abs(z))
        acc = t[:, 0:128]
        for c in range(1, B // 128):
            acc = acc * t[:, c * 128:(c + 1) * 128]
        part = jnp.sum(m) + jnp.sum(jnp.log(acc))

        @pl.when(i == 0)
        def _():
            out_ref[...] = jnp.zeros_like(out_ref)

        out_ref[...] += jnp.reshape(part, (1, 1))

    out = pl.pallas_call(
        body,
        grid=(B // RB,),
        in_specs=[
            pl.BlockSpec((RB, 1), lambda i: (i, 0)),
            pl.BlockSpec((1, B), lambda i: (0, 0)),
        ],
        out_specs=pl.BlockSpec((1, 1), lambda i: (0, 0)),
        out_shape=jax.ShapeDtypeStruct((1, 1), jnp.float32),
    )(p2, n2)
    return out[0, 0]


B_SC = 3584


def kernel(user, pos_item, neg_item, user_table, item_table):
    nflat = jnp.reshape(neg_item, (-1,))
    ttu, tti = user_table.T, item_table.T
    p_sc, n_sc = _sc_scores(user[:B_SC], pos_item[:B_SC], nflat[:B_SC],
                            ttu, tti)
    p_tc, n_tc = _tc_scores(user[B_SC:], pos_item[B_SC:], nflat[B_SC:],
                            ttu, tti)
    p = jnp.concatenate([p_sc, p_tc[:, 0]])
    nvec = jnp.concatenate([n_sc, n_tc[:, 0]])
    loss = _tc_loss_sum(p, nvec) / (B * B)
    logits = p.reshape(B, 1)
    return (logits, loss)


# speedup vs baseline: 1.9314x; 1.0736x over previous
import functools

import jax
import jax.numpy as jnp
from jax import lax
from jax.experimental import pallas as pl
from jax.experimental.pallas import tpu as pltpu
from jax.experimental.pallas import tpu_sc as plsc

B = 4096
D = 64
NROWS = 1000000
L = 16
NC = 2
NS = 16
NW = NC * NS
BPW = B // NW
NG = BPW // L
W = 128
WMAX = NROWS - W

RB = 512


def _sc_scores(user, pos_item, neg_item, user_table_t, item_table_t):
    BS = user.shape[0]
    BPW = BS // NW
    NG = BPW // L
    mesh = plsc.VectorSubcoreMesh(core_axis_name="c", subcore_axis_name="s")

    @functools.partial(
        pl.kernel,
        mesh=mesh,
        compiler_params=pltpu.CompilerParams(needs_layout_passes=False),
        out_type=[
            jax.ShapeDtypeStruct((BS,), jnp.float32),
            jax.ShapeDtypeStruct((BS,), jnp.float32),
        ],
        scratch_types=[
            pltpu.VMEM((BPW,), jnp.int32),
            pltpu.VMEM((BPW,), jnp.int32),
            pltpu.VMEM((BPW,), jnp.int32),
            pltpu.VMEM((D, W), jnp.float32),
            pltpu.VMEM((D, W), jnp.float32),
            pltpu.VMEM((D, W), jnp.float32),
            pltpu.VMEM((D, W), jnp.float32),
            pltpu.VMEM((D, W), jnp.float32),
            pltpu.VMEM((D, W), jnp.float32),
            pltpu.VMEM((D, W), jnp.float32),
            pltpu.VMEM((D, W), jnp.float32),
            pltpu.VMEM((D, W), jnp.float32),
            pltpu.VMEM((BPW,), jnp.float32),
            pltpu.VMEM((BPW,), jnp.float32),
            pltpu.SemaphoreType.DMA,
            pltpu.SemaphoreType.DMA,
            pltpu.SemaphoreType.DMA,
        ],
    )
    def k(u_hbm, p_hbm, n_hbm, ut_hbm, it_hbm, ps_out, ns_out,
          uidx, pidx, nidx, wu0, wu1, wu2, wp0, wp1, wp2, wn0, wn1, wn2,
          ps_v, ns_v, sem0, sem1, sem2):
        wid = lax.axis_index("s") * NC + lax.axis_index("c")
        base = wid * BPW
        pltpu.sync_copy(u_hbm.at[pl.ds(base, BPW)], uidx)
        pltpu.sync_copy(p_hbm.at[pl.ds(base, BPW)], pidx)
        pltpu.sync_copy(n_hbm.at[pl.ds(base, BPW)], nidx)

        lane = lax.iota(jnp.int32, L)
        rows4 = [lax.iota(jnp.int32, L) + c * L for c in range(D // L)]

        def win_off(i):
            return i & jnp.int32(-W)

        bufs = ((wu0, wp0, wn0, sem0), (wu1, wp1, wn1, sem1),
                (wu2, wp2, wn2, sem2))
        NBUF = len(bufs)

        def group(g, carry):
            gbase = g * L
            uvec = uidx[pl.ds(gbase, L)]
            pvec = pidx[pl.ds(gbase, L)]
            nvec = nidx[pl.ds(gbase, L)]

            def issue(jj):
                bu, bp, bn, sem = bufs[jj % NBUF]
                iu, ip, iN = uvec[jj], pvec[jj], nvec[jj]
                ou, op, oN = win_off(iu), win_off(ip), win_off(iN)
                cs = (
                    pltpu.async_copy(
                        ut_hbm.at[:, pl.ds(pl.multiple_of(ou, W), W)], bu, sem),
                    pltpu.async_copy(
                        it_hbm.at[:, pl.ds(pl.multiple_of(op, W), W)], bp, sem),
                    pltpu.async_copy(
                        it_hbm.at[:, pl.ds(pl.multiple_of(oN, W), W)], bn, sem),
                )
                return cs, (iu - ou, ip - op, iN - oN)

            psel = jnp.zeros((L,), jnp.float32)
            nsel = jnp.zeros((L,), jnp.float32)
            inflight = [issue(0), issue(1)]
            for j in range(L):
                if j + 2 < L:
                    inflight.append(issue(j + 2))
                cs, (lu_s, lp_s, ln_s) = inflight.pop(0)
                for c_ in cs:
                    c_.wait()
                bu, bp, bn, _ = bufs[j % NBUF]
                lu = jnp.full((L,), lu_s, jnp.int32)
                lp = jnp.full((L,), lp_s, jnp.int32)
                ln = jnp.full((L,), ln_s, jnp.int32)
                pacc = jnp.zeros((L,), jnp.float32)
                nacc = jnp.zeros((L,), jnp.float32)
                for c in range(D // L):
                    uv = plsc.load_gather(bu, [rows4[c], lu])
                    pv = plsc.load_gather(bp, [rows4[c], lp])
                    nv = plsc.load_gather(bn, [rows4[c], ln])
                    pacc = pacc + uv * pv
                    nacc = nacc + uv * nv
                mask = lane == j
                psel = jnp.where(mask, jnp.sum(pacc), psel)
                nsel = jnp.where(mask, jnp.sum(nacc), nsel)
            ps_v[pl.ds(gbase, L)] = psel
            ns_v[pl.ds(gbase, L)] = nsel
            return carry

        lax.fori_loop(0, NG, group, 0)
        pltpu.sync_copy(ps_v, ps_out.at[pl.ds(base, BPW)])
        pltpu.sync_copy(ns_v, ns_out.at[pl.ds(base, BPW)])

    return k(user, pos_item, neg_item, user_table_t, item_table_t)


K = 8


def _tc_scores(user, pos_item, neg_item, user_table_t, item_table_t):
    S = user.shape[0]

    def body(uref, pref, nref, *refs):
        ubs, pbs, nbs = refs[0:K], refs[K:2 * K], refs[2 * K:3 * K]
        po, no = refs[3 * K], refs[3 * K + 1]
        i = pl.program_id(0)
        lanes = lax.broadcasted_iota(jnp.int32, (1, W), 1)
        pouts, nouts = [], []
        for k in range(K):
            r = i * K + k
            lu, lp, ln = uref[r] & (W - 1), pref[r] & (W - 1), nref[r] & (W - 1)
            uv = jnp.sum(jnp.where(lanes == lu, ubs[k][...], 0.0),
                         axis=1, keepdims=True)
            pv = jnp.sum(jnp.where(lanes == lp, pbs[k][...], 0.0),
                         axis=1, keepdims=True)
            nv = jnp.sum(jnp.where(lanes == ln, nbs[k][...], 0.0),
                         axis=1, keepdims=True)
            pouts.append(jnp.reshape(jnp.sum(uv * pv), (1, 1)))
            nouts.append(jnp.reshape(jnp.sum(uv * nv), (1, 1)))
        po[...] = jnp.concatenate(pouts, axis=0)
        no[...] = jnp.concatenate(nouts, axis=0)

    def uspec(k):
        return pl.BlockSpec((D, W), lambda i, u, p, n, k=k: (0, u[i * K + k] // W))

    def pspec(k):
        return pl.BlockSpec((D, W), lambda i, u, p, n, k=k: (0, p[i * K + k] // W))

    def nspec(k):
        return pl.BlockSpec((D, W), lambda i, u, p, n, k=k: (0, n[i * K + k] // W))

    grid_spec = pltpu.PrefetchScalarGridSpec(
        num_scalar_prefetch=3,
        grid=(S // K,),
        in_specs=([uspec(k) for k in range(K)]
                  + [pspec(k) for k in range(K)]
                  + [nspec(k) for k in range(K)]),
        out_specs=[
            pl.BlockSpec((K, 1), lambda i, u, p, n: (i, 0)),
            pl.BlockSpec((K, 1), lambda i, u, p, n: (i, 0)),
        ],
    )
    ops = ([user_table_t] * K) + ([item_table_t] * (2 * K))
    return pl.pallas_call(
        body,
        grid_spec=grid_spec,
        out_shape=[jax.ShapeDtypeStruct((S, 1), jnp.float32)] * 2,
    )(user, pos_item, neg_item, *ops)


def _tc_loss_sum(p, n):
    p2 = p.reshape(B, 1)
    n2 = n.reshape(1, B)

    def body(p_ref, n_ref, out_ref):
        i = pl.program_id(0)
        z = n_ref[...] - p_ref[...]
        a = jnp.abs(z)
        t = 1.0 + jnp.exp(-a)
        acc = t[:, 0:128]
        for c in range(1, B // 128):
            acc = acc * t[:, c * 128:(c + 1) * 128]
        sum_z = RB * jnp.sum(n_ref[...]) - B * jnp.sum(p_ref[...])
        part = 0.5 * (sum_z + jnp.sum(a)) + jnp.sum(jnp.log(acc))

        @pl.when(i == 0)
        def _():
            out_ref[...] = jnp.zeros_like(out_ref)

        out_ref[...] += jnp.reshape(part, (1, 1))

    out = pl.pallas_call(
        body,
        grid=(B // RB,),
        in_specs=[
            pl.BlockSpec((RB, 1), lambda i: (i, 0)),
            pl.BlockSpec((1, B), lambda i: (0, 0)),
        ],
        out_specs=pl.BlockSpec((1, 1), lambda i: (0, 0)),
        out_shape=jax.ShapeDtypeStruct((1, 1), jnp.float32),
    )(p2, n2)
    return out[0, 0]


B_SC = 3584


def kernel(user, pos_item, neg_item, user_table, item_table):
    nflat = jnp.reshape(neg_item, (-1,))
    ttu, tti = user_table.T, item_table.T
    p_sc, n_sc = _sc_scores(user[:B_SC], pos_item[:B_SC], nflat[:B_SC],
                            ttu, tti)
    p_tc, n_tc = _tc_scores(user[B_SC:], pos_item[B_SC:], nflat[B_SC:],
                            ttu, tti)
    p = jnp.concatenate([p_sc, p_tc[:, 0]])
    nvec = jnp.concatenate([n_sc, n_tc[:, 0]])
    loss = _tc_loss_sum(p, nvec) / (B * B)
    logits = p.reshape(B, 1)
    return (logits, loss)
